# baseline (speedup 1.0000x reference)
import functools

import jax
import jax.numpy as jnp
from jax import lax
from jax.experimental import pallas as pl
from jax.experimental.pallas import tpu as pltpu
from jax.experimental.pallas import tpu_sc as plsc

_BATCH = 16384
_F_GMF = 16
_D_MLP = 64
_NC = 2
_NS = 16
_NW = _NC * _NS
_BPW = _BATCH // _NW
_CHUNK = 128
_NCHUNK = _BPW // _CHUNK


def _sc_gather_body(user_hbm, item_hbm, ug_hbm, ig_hbm, um_hbm, im_hbm,
                    out_gu, out_gi, out_mu, out_mi,
                    idx_u, idx_i, gu_v, gi_v, mu_v, mi_v, sem):
    wid = lax.axis_index("s") * _NC + lax.axis_index("c")
    base = wid * _BPW
    pltpu.sync_copy(user_hbm.at[pl.ds(base, _BPW)], idx_u)
    pltpu.sync_copy(item_hbm.at[pl.ds(base, _BPW)], idx_i)
    copies = []
    for j in range(_NCHUNK):
        s = pl.ds(j * _CHUNK, _CHUNK)
        copies.append(pltpu.async_copy(ug_hbm.at[idx_u.at[s]], gu_v.at[s], sem))
        copies.append(pltpu.async_copy(ig_hbm.at[idx_i.at[s]], gi_v.at[s], sem))
        copies.append(pltpu.async_copy(um_hbm.at[idx_u.at[s]], mu_v.at[s], sem))
        copies.append(pltpu.async_copy(im_hbm.at[idx_i.at[s]], mi_v.at[s], sem))
    for c in copies:
        c.wait()
    pltpu.sync_copy(gu_v, out_gu.at[pl.ds(base, _BPW)])
    pltpu.sync_copy(gi_v, out_gi.at[pl.ds(base, _BPW)])
    pltpu.sync_copy(mu_v, out_mu.at[pl.ds(base, _BPW)])
    pltpu.sync_copy(mi_v, out_mi.at[pl.ds(base, _BPW)])


_sc_gather = functools.partial(
    pl.kernel,
    mesh=plsc.VectorSubcoreMesh(core_axis_name="c", subcore_axis_name="s"),
    out_type=[
        jax.ShapeDtypeStruct((_BATCH, _F_GMF), jnp.float32),
        jax.ShapeDtypeStruct((_BATCH, _F_GMF), jnp.float32),
        jax.ShapeDtypeStruct((_BATCH, _D_MLP), jnp.float32),
        jax.ShapeDtypeStruct((_BATCH, _D_MLP), jnp.float32),
    ],
    scratch_types=[
        pltpu.VMEM((_BPW,), jnp.int32),
        pltpu.VMEM((_BPW,), jnp.int32),
        pltpu.VMEM((_BPW, _F_GMF), jnp.float32),
        pltpu.VMEM((_BPW, _F_GMF), jnp.float32),
        pltpu.VMEM((_BPW, _D_MLP), jnp.float32),
        pltpu.VMEM((_BPW, _D_MLP), jnp.float32),
        pltpu.SemaphoreType.DMA,
    ],
    compiler_params=pltpu.CompilerParams(use_tc_tiling_on_sc=False),
)(_sc_gather_body)


_BB = 2048


def _dense_body(gu_ref, gi_ref, mu_ref, mi_ref,
                w1u_ref, w1i_ref, b1_ref, w2_ref, b2_ref, w3_ref, b3_ref,
                wnm_ref, wng_ref, bn_ref, out_ref):
    h = jnp.dot(mu_ref[...], w1u_ref[...], preferred_element_type=jnp.float32)
    h = h + jnp.dot(mi_ref[...], w1i_ref[...], preferred_element_type=jnp.float32)
    h = jnp.maximum(h + b1_ref[...], 0.0)
    h = jnp.maximum(
        jnp.dot(h, w2_ref[...], preferred_element_type=jnp.float32) + b2_ref[...], 0.0)
    m = jnp.maximum(
        jnp.dot(h, w3_ref[...], preferred_element_type=jnp.float32) + b3_ref[...], 0.0)
    g = gu_ref[...] * gi_ref[...]
    s = (jnp.sum(m * wnm_ref[...], axis=1, keepdims=True)
         + jnp.sum(g * wng_ref[...], axis=1, keepdims=True) + bn_ref[0, 0])
    out_ref[...] = 1.0 / (1.0 + jnp.exp(-s))


def _dense(gu, gi, mu, mi, w1u, w1i, b1, w2, b2, w3, b3, wnm, wng, bn):
    grid = _BATCH // _BB
    row_spec_g = pl.BlockSpec((_BB, _F_GMF), lambda i: (i, 0))
    row_spec_m = pl.BlockSpec((_BB, _D_MLP), lambda i: (i, 0))

    def full(x):
        return pl.BlockSpec(x.shape, lambda i: tuple(0 for _ in x.shape))

    return pl.pallas_call(
        _dense_body,
        grid=(grid,),
        in_specs=[row_spec_g, row_spec_g, row_spec_m, row_spec_m,
                  full(w1u), full(w1i), full(b1), full(w2), full(b2),
                  full(w3), full(b3), full(wnm), full(wng), full(bn)],
        out_specs=pl.BlockSpec((_BB, 1), lambda i: (i, 0)),
        out_shape=jax.ShapeDtypeStruct((_BATCH, 1), jnp.float32),
    )(gu, gi, mu, mi, w1u, w1i, b1, w2, b2, w3, b3, wnm, wng, bn)


@jax.jit
def kernel(user, item, user_embed_GMF, item_embed_GMF, user_embed_MLP,
           item_embed_MLP, W1, b1, W2, b2, W3, b3, Wn, bn):
    user = user.astype(jnp.int32)
    item = item.astype(jnp.int32)
    gu, gi, mu, mi = _sc_gather(user, item, user_embed_GMF, item_embed_GMF,
                                user_embed_MLP, item_embed_MLP)
    w1u = W1[:, :_D_MLP].T
    w1i = W1[:, _D_MLP:].T
    w2 = W2.T
    w3 = W3.T
    wnm = Wn[:, :_F_GMF]
    wng = Wn[:, _F_GMF:]
    return _dense(gu, gi, mu, mi, w1u, w1i, b1.reshape(1, -1), w2,
                  b2.reshape(1, -1), w3, b3.reshape(1, -1), wnm, wng,
                  bn.reshape(1, 1))
